# baseline (speedup 1.0000x reference)
import functools

import jax
import jax.numpy as jnp
from jax.experimental import pallas as pl
from jax.experimental.pallas import tpu as pltpu

_MM_DTYPE = jnp.bfloat16
_CP = 512
_G = 4


def _hebbian_body(scal_ref, x_ref, w_ref, o_ref,
                  S_scr, prev_scr, M_scr, *, CP, G):
    p = pl.program_id(0)
    c = pl.program_id(1)
    log_gamma = scal_ref[0]
    gCC = scal_ref[1]

    @pl.when(jnp.logical_and(p == 0, c == 0))
    def _init_mask():
        r = jax.lax.broadcasted_iota(jnp.int32, (CP, CP), 0).astype(jnp.float32)
        k = jax.lax.broadcasted_iota(jnp.int32, (CP, CP), 1).astype(jnp.float32)
        M_scr[...] = jnp.where(r > k, jnp.exp((r - 1.0 - k) * log_gamma), 0.0)

    @pl.when(c == 0)
    def _init_state():
        S_scr[...] = jnp.zeros_like(S_scr)
        prev_scr[...] = jnp.zeros_like(prev_scr)

    ci = jax.lax.broadcasted_iota(jnp.int32, (CP, 1), 0).astype(jnp.float32)
    gp = jnp.exp(ci * log_gamma).astype(_MM_DTYPE)
    gw = jnp.exp(((CP - 1.0) - ci) * log_gamma).astype(_MM_DTYPE)
    ww = w_ref[0]
    wr = w_ref[1]
    M = M_scr[...]

    for g in range(G):
        x = x_ref[0, g]
        xb = x.astype(_MM_DTYPE)
        wk = jnp.concatenate([prev_scr[g:g + 1], x[:CP - 1]], axis=0)
        prev_scr[g:g + 1] = x[CP - 1:CP]
        wkb = wk.astype(_MM_DTYPE)

        v = jnp.dot(xb, ww, preferred_element_type=jnp.float32)
        vb = v.astype(_MM_DTYPE)

        sa = jax.lax.dot_general(xb, wkb, (((1,), (1,)), ((), ())),
                                 preferred_element_type=jnp.float32)

        reads = (jnp.dot(xb * gp, S_scr[g], preferred_element_type=jnp.float32)
                 + jnp.dot((sa * M).astype(_MM_DTYPE), vb,
                           preferred_element_type=jnp.float32))
        o_ref[0, g] = x + jnp.dot(reads.astype(_MM_DTYPE), wr,
                                  preferred_element_type=jnp.float32)

        vg = vb * gw
        upd = jax.lax.dot_general(
            wkb, vg, (((0,), (0,)), ((), ())),
            preferred_element_type=jnp.float32)
        S_scr[g] = gCC.astype(_MM_DTYPE) * S_scr[g] + upd.astype(_MM_DTYPE)


def kernel(out, W_write, W_read, decay, log_alpha):
    B, T, D = out.shape
    CP, G = _CP, _G
    NB = T // CP

    gamma = jax.nn.sigmoid(decay)
    log_gamma = jnp.log(gamma)
    scal = jnp.stack([log_gamma, jnp.exp(CP * log_gamma)]).astype(jnp.float32)

    out_f = out.astype(jnp.float32).reshape(B // G, G, T, D)
    w2 = jnp.stack([W_write, W_read * jnp.exp(log_alpha)]
                   ).astype(_MM_DTYPE).transpose(0, 2, 1)

    res = pl.pallas_call(
        functools.partial(_hebbian_body, CP=CP, G=G),
        grid=(B // G, NB),
        in_specs=[
            pl.BlockSpec(memory_space=pltpu.SMEM),
            pl.BlockSpec((1, G, CP, D), lambda p, c: (p, 0, c, 0)),
            pl.BlockSpec((2, D, D), lambda p, c: (0, 0, 0)),
        ],
        out_specs=pl.BlockSpec((1, G, CP, D), lambda p, c: (p, 0, c, 0)),
        out_shape=jax.ShapeDtypeStruct((B // G, G, T, D), jnp.float32),
        scratch_shapes=[
            pltpu.VMEM((G, D, D), _MM_DTYPE),
            pltpu.VMEM((G, D), jnp.float32),
            pltpu.VMEM((CP, CP), jnp.float32),
        ],
        compiler_params=pltpu.CompilerParams(
            dimension_semantics=("arbitrary", "arbitrary"),
            vmem_limit_bytes=60 * 1024 * 1024,
        ),
    )(scal, out_f, w2)
    return res.reshape(B, T, D).astype(out.dtype)


# speedup vs baseline: 1.0106x; 1.0106x over previous
import functools

import jax
import jax.numpy as jnp
from jax.experimental import pallas as pl
from jax.experimental.pallas import tpu as pltpu

_MM_DTYPE = jnp.bfloat16
_CP = 512
_G = 2


def _hebbian_body(scal_ref, x_ref, ww_ref, wr_ref, o_ref,
                  S_scr, prev_scr, M_scr, *, CP, G):
    c = pl.program_id(1)
    log_gamma = scal_ref[0]
    gCC = scal_ref[1]

    @pl.when(c == 0)
    def _init():
        S_scr[...] = jnp.zeros_like(S_scr)
        prev_scr[...] = jnp.zeros_like(prev_scr)
        r = jax.lax.broadcasted_iota(jnp.int32, (CP, CP), 0).astype(jnp.float32)
        k = jax.lax.broadcasted_iota(jnp.int32, (CP, CP), 1).astype(jnp.float32)
        M_scr[...] = jnp.where(r > k, jnp.exp((r - 1.0 - k) * log_gamma), 0.0)

    ci = jax.lax.broadcasted_iota(jnp.int32, (CP, 1), 0).astype(jnp.float32)
    gp = jnp.exp(ci * log_gamma).astype(_MM_DTYPE)
    gw = jnp.exp(((CP - 1.0) - ci) * log_gamma).astype(_MM_DTYPE)
    ww = ww_ref[...]
    wr = wr_ref[...]
    M = M_scr[...]

    for g in range(G):
        x = x_ref[0, g]
        xb = x.astype(_MM_DTYPE)
        wk = jnp.concatenate([prev_scr[g:g + 1], x[:CP - 1]], axis=0)
        prev_scr[g:g + 1] = x[CP - 1:CP]
        wkb = wk.astype(_MM_DTYPE)

        v = jnp.dot(xb, ww, preferred_element_type=jnp.float32)
        vb = v.astype(_MM_DTYPE)

        sa = jax.lax.dot_general(xb, wkb, (((1,), (1,)), ((), ())),
                                 preferred_element_type=jnp.float32)

        reads = (jnp.dot(xb * gp, S_scr[g], preferred_element_type=jnp.float32)
                 + jnp.dot((sa * M).astype(_MM_DTYPE), vb,
                           preferred_element_type=jnp.float32))
        o_ref[0, g] = x + jnp.dot(reads.astype(_MM_DTYPE), wr,
                                  preferred_element_type=jnp.float32)

        vg = vb * gw
        upd = jax.lax.dot_general(
            wkb, vg, (((0,), (0,)), ((), ())),
            preferred_element_type=jnp.float32)
        S_scr[g] = gCC.astype(_MM_DTYPE) * S_scr[g] + upd.astype(_MM_DTYPE)


def kernel(out, W_write, W_read, decay, log_alpha):
    B, T, D = out.shape
    CP, G = _CP, _G
    NB = T // CP

    gamma = jax.nn.sigmoid(decay)
    log_gamma = jnp.log(gamma)
    scal = jnp.stack([log_gamma, jnp.exp(CP * log_gamma)]).astype(jnp.float32)

    out_f = out.astype(jnp.float32).reshape(B // G, G, T, D)
    wwT = W_write.T.astype(_MM_DTYPE)
    wrT = (W_read * jnp.exp(log_alpha)).T.astype(_MM_DTYPE)

    res = pl.pallas_call(
        functools.partial(_hebbian_body, CP=CP, G=G),
        grid=(B // G, NB),
        in_specs=[
            pl.BlockSpec(memory_space=pltpu.SMEM),
            pl.BlockSpec((1, G, CP, D), lambda p, c: (p, 0, c, 0)),
            pl.BlockSpec((D, D), lambda p, c: (0, 0)),
            pl.BlockSpec((D, D), lambda p, c: (0, 0)),
        ],
        out_specs=pl.BlockSpec((1, G, CP, D), lambda p, c: (p, 0, c, 0)),
        out_shape=jax.ShapeDtypeStruct((B // G, G, T, D), jnp.float32),
        scratch_shapes=[
            pltpu.VMEM((G, D, D), _MM_DTYPE),
            pltpu.VMEM((G, D), jnp.float32),
            pltpu.VMEM((CP, CP), jnp.float32),
        ],
        compiler_params=pltpu.CompilerParams(
            dimension_semantics=("parallel", "arbitrary"),
            vmem_limit_bytes=60 * 1024 * 1024,
        ),
    )(scal, out_f, wwT, wrT)
    return res.reshape(B, T, D).astype(out.dtype)


# speedup vs baseline: 1.0246x; 1.0139x over previous
import functools

import jax
import jax.numpy as jnp
from jax.experimental import pallas as pl
from jax.experimental.pallas import tpu as pltpu

_MM_DTYPE = jnp.bfloat16
_CP = 512
_G = 2


def _hebbian_body(scal_ref, x_ref, ww_ref, wr_ref, o_ref,
                  S_scr, prev_scr, M_scr, *, CP, G):
    c = pl.program_id(1)
    log_gamma = scal_ref[0]
    gCC = scal_ref[1]

    @pl.when(c == 0)
    def _init():
        S_scr[...] = jnp.zeros_like(S_scr)
        prev_scr[...] = jnp.zeros_like(prev_scr)
        r = jax.lax.broadcasted_iota(jnp.int32, (CP, CP), 0).astype(jnp.float32)
        k = jax.lax.broadcasted_iota(jnp.int32, (CP, CP), 1).astype(jnp.float32)
        M_scr[...] = jnp.where(r > k, jnp.exp((r - 1.0 - k) * log_gamma), 0.0)

    ci = jax.lax.broadcasted_iota(jnp.int32, (CP, 1), 0).astype(jnp.float32)
    gp = jnp.exp(ci * log_gamma).astype(_MM_DTYPE)
    gw = jnp.exp(((CP - 1.0) - ci) * log_gamma).astype(_MM_DTYPE)
    ww = ww_ref[...]
    wr = wr_ref[...]
    M = M_scr[...]

    for g in range(G):
        x = x_ref[0, g]
        xb = x.astype(_MM_DTYPE)
        wk = jnp.concatenate([prev_scr[g:g + 1], x[:CP - 1]], axis=0)
        prev_scr[g:g + 1] = x[CP - 1:CP]
        wkb = wk.astype(_MM_DTYPE)

        v = jax.lax.dot_general(xb, ww, (((1,), (1,)), ((), ())),
                                preferred_element_type=jnp.float32)
        vb = v.astype(_MM_DTYPE)

        sa = jax.lax.dot_general(xb, wkb, (((1,), (1,)), ((), ())),
                                 preferred_element_type=jnp.float32)

        reads = (jnp.dot(xb * gp, S_scr[g], preferred_element_type=jnp.float32)
                 + jnp.dot((sa * M).astype(_MM_DTYPE), vb,
                           preferred_element_type=jnp.float32))
        o_ref[0, g] = x + jax.lax.dot_general(
            reads.astype(_MM_DTYPE), wr, (((1,), (1,)), ((), ())),
            preferred_element_type=jnp.float32)

        vg = vb * gw
        upd = jax.lax.dot_general(
            wkb, vg, (((0,), (0,)), ((), ())),
            preferred_element_type=jnp.float32)
        S_scr[g] = gCC.astype(_MM_DTYPE) * S_scr[g] + upd.astype(_MM_DTYPE)


def kernel(out, W_write, W_read, decay, log_alpha):
    B, T, D = out.shape
    CP, G = _CP, _G
    NB = T // CP

    gamma = jax.nn.sigmoid(decay)
    log_gamma = jnp.log(gamma)
    scal = jnp.stack([log_gamma, jnp.exp(CP * log_gamma)]).astype(jnp.float32)

    out_f = out.astype(jnp.float32).reshape(B // G, G, T, D)
    wwT = W_write.astype(_MM_DTYPE)
    wrT = (W_read * jnp.exp(log_alpha)).astype(_MM_DTYPE)

    res = pl.pallas_call(
        functools.partial(_hebbian_body, CP=CP, G=G),
        grid=(B // G, NB),
        in_specs=[
            pl.BlockSpec(memory_space=pltpu.SMEM),
            pl.BlockSpec((1, G, CP, D), lambda p, c: (p, 0, c, 0)),
            pl.BlockSpec((D, D), lambda p, c: (0, 0)),
            pl.BlockSpec((D, D), lambda p, c: (0, 0)),
        ],
        out_specs=pl.BlockSpec((1, G, CP, D), lambda p, c: (p, 0, c, 0)),
        out_shape=jax.ShapeDtypeStruct((B // G, G, T, D), jnp.float32),
        scratch_shapes=[
            pltpu.VMEM((G, D, D), _MM_DTYPE),
            pltpu.VMEM((G, D), jnp.float32),
            pltpu.VMEM((CP, CP), jnp.float32),
        ],
        compiler_params=pltpu.CompilerParams(
            dimension_semantics=("parallel", "arbitrary"),
            vmem_limit_bytes=60 * 1024 * 1024,
        ),
    )(scal, out_f, wwT, wrT)
    return res.reshape(B, T, D).astype(out.dtype)
